# Initial kernel scaffold; baseline (speedup 1.0000x reference)
import jax
import jax.numpy as jnp
from jax.experimental import pallas as pl


def kernel(node_feat, score, codebook, W1, b1, gamma, beta, W2, b2, batch):
    raise NotImplementedError("write your pallas kernel here")


# speedup vs baseline: 3.1566x; 3.1566x over previous
import functools

import jax
import jax.numpy as jnp
from jax import lax
from jax.experimental import pallas as pl
from jax.experimental.pallas import tpu as pltpu

G = 1024
COMMIT = 0.25


def _vq_pool_body(x_ref, sc_ref, b_ref, cb_ref, acc_ref, cnt_ref, cmt_ref):
    i = pl.program_id(0)

    @pl.when(i == 0)
    def _init():
        acc_ref[...] = jnp.zeros_like(acc_ref)
        cnt_ref[...] = jnp.zeros_like(cnt_ref)
        cmt_ref[...] = jnp.zeros_like(cmt_ref)

    x = x_ref[...]
    cb = cb_ref[...]
    K = cb.shape[0]

    x_sq = jnp.sum(x * x, axis=1, keepdims=True)
    cb_sq = jnp.sum(cb * cb, axis=1)[None, :]
    xc = lax.dot_general(x, cb, (((1,), (1,)), ((), ())),
                         preferred_element_type=jnp.float32)
    dist = x_sq - 2.0 * xc + cb_sq

    m = jnp.min(dist, axis=1, keepdims=True)
    iota_k = lax.broadcasted_iota(jnp.int32, dist.shape, 1)
    idx = jnp.min(jnp.where(dist == m, iota_k, K), axis=1, keepdims=True)

    cmt_ref[...] += jnp.sum(m).reshape(1, 1)

    onehot_k = (iota_k == idx).astype(jnp.float32)
    q = lax.dot_general(onehot_k, cb, (((1,), (0,)), ((), ())),
                        preferred_element_type=jnp.float32)

    score = sc_ref[...]
    r = x + q
    c = r * score
    cr = jnp.concatenate([c, r], axis=1)

    seg = b_ref[...]
    iota_g = lax.broadcasted_iota(jnp.int32, (seg.shape[0], G), 1)
    onehot_g = (iota_g == seg).astype(jnp.float32)

    acc_ref[...] += lax.dot_general(onehot_g, cr, (((0,), (0,)), ((), ())),
                                    preferred_element_type=jnp.float32)
    cnt_ref[...] += jnp.sum(onehot_g, axis=0)[:, None]


def _classifier_body(acc_ref, cnt_ref, cmt_ref, w1_ref, b1_ref, g_ref, be_ref,
                     w2_ref, b2_ref, logit_ref, cg_ref, sg_ref, cmt_out_ref,
                     *, n_total, d):
    acc = acc_ref[...]
    denom = jnp.maximum(cnt_ref[...], 1.0)
    cg = acc[:, :d] / denom
    rg = acc[:, d:] / denom
    sg = rg - cg
    cg_ref[...] = cg
    sg_ref[...] = sg

    h = lax.dot_general(cg, w1_ref[...], (((1,), (0,)), ((), ())),
                        preferred_element_type=jnp.float32) + b1_ref[...]
    gn = h.shape[0]
    mu = jnp.sum(h, axis=0, keepdims=True) / gn
    var = jnp.sum((h - mu) ** 2, axis=0, keepdims=True) / gn
    hn = (h - mu) / jnp.sqrt(var + 1e-5) * g_ref[...] + be_ref[...]
    hr = jnp.maximum(hn, 0.0)
    logit_ref[...] = lax.dot_general(hr, w2_ref[...], (((1,), (0,)), ((), ())),
                                     preferred_element_type=jnp.float32) + b2_ref[...]

    cmt_out_ref[...] = cmt_ref[...] * (COMMIT / (n_total * d))


def kernel(node_feat, score, codebook, W1, b1, gamma, beta, W2, b2, batch):
    N, D = node_feat.shape
    K = codebook.shape[0]
    H = W1.shape[1]

    B = 2000
    n_blocks = N // B
    assert n_blocks * B == N

    batch2d = batch.astype(jnp.int32).reshape(N, 1)

    acc, cnt, cmt = pl.pallas_call(
        _vq_pool_body,
        grid=(n_blocks,),
        in_specs=[
            pl.BlockSpec((B, D), lambda i: (i, 0)),
            pl.BlockSpec((B, 1), lambda i: (i, 0)),
            pl.BlockSpec((B, 1), lambda i: (i, 0)),
            pl.BlockSpec((K, D), lambda i: (0, 0)),
        ],
        out_specs=[
            pl.BlockSpec((G, 2 * D), lambda i: (0, 0)),
            pl.BlockSpec((G, 1), lambda i: (0, 0)),
            pl.BlockSpec((1, 1), lambda i: (0, 0)),
        ],
        out_shape=[
            jax.ShapeDtypeStruct((G, 2 * D), jnp.float32),
            jax.ShapeDtypeStruct((G, 1), jnp.float32),
            jax.ShapeDtypeStruct((1, 1), jnp.float32),
        ],
    )(node_feat, score, batch2d, codebook)

    logit, cg, sg, cmt_out = pl.pallas_call(
        functools.partial(_classifier_body, n_total=N, d=D),
        in_specs=[pl.BlockSpec(s.shape, lambda: (0,) * len(s.shape)) for s in (
            jax.ShapeDtypeStruct((G, 2 * D), jnp.float32),
            jax.ShapeDtypeStruct((G, 1), jnp.float32),
            jax.ShapeDtypeStruct((1, 1), jnp.float32),
            jax.ShapeDtypeStruct((D, H), jnp.float32),
            jax.ShapeDtypeStruct((1, H), jnp.float32),
            jax.ShapeDtypeStruct((1, H), jnp.float32),
            jax.ShapeDtypeStruct((1, H), jnp.float32),
            jax.ShapeDtypeStruct((H, 1), jnp.float32),
            jax.ShapeDtypeStruct((1, 1), jnp.float32),
        )],
        out_specs=[
            pl.BlockSpec((G, 1), lambda: (0, 0)),
            pl.BlockSpec((G, D), lambda: (0, 0)),
            pl.BlockSpec((G, D), lambda: (0, 0)),
            pl.BlockSpec((1, 1), lambda: (0, 0)),
        ],
        out_shape=[
            jax.ShapeDtypeStruct((G, 1), jnp.float32),
            jax.ShapeDtypeStruct((G, D), jnp.float32),
            jax.ShapeDtypeStruct((G, D), jnp.float32),
            jax.ShapeDtypeStruct((1, 1), jnp.float32),
        ],
    )(acc, cnt, cmt, W1, b1.reshape(1, H), gamma.reshape(1, H),
      beta.reshape(1, H), W2, b2.reshape(1, 1))

    return (logit, cg, sg, cmt_out.reshape(()))
